# baseline (speedup 1.0000x reference)
import functools

import jax
import jax.numpy as jnp
from jax import lax
from jax.experimental import pallas as pl
from jax.experimental.pallas import tpu as pltpu
from jax.experimental.pallas import tpu_sc as plsc

N_NODES = 10000
D_FEAT = 128
HIDDEN = 16
N_ACTIONS = 4
N_EDGES = 320000

NC = 2
NS = 16
NW = NC * NS
EPW = N_EDGES // NW
CHUNK = 125
NCHUNK = EPW // CHUNK
NBUF = 10
DCHUNK = 80
DNCHUNK = EPW // DCHUNK
DEG_FIRE = 5
ROWS_PER_SUB = 624
ROWS_TAIL = N_NODES - NS * ROWS_PER_SUB

PACK = 8
NP = N_NODES // PACK
FLAT = N_NODES * HIDDEN

_HIGH = jax.lax.Precision.HIGHEST

_mesh = plsc.VectorSubcoreMesh(core_axis_name="c", subcore_axis_name="s")
_sc_params = pltpu.CompilerParams(use_tc_tiling_on_sc=False)


def _zero_acc(zeros_hbm, acc_sh, s):
    rbase = s * ROWS_PER_SUB
    tbase = NS * ROWS_PER_SUB
    pltpu.sync_copy(zeros_hbm.at[pl.ds(rbase, ROWS_PER_SUB)],
                    acc_sh.at[pl.ds(rbase, ROWS_PER_SUB)])

    @pl.when(s == 0)
    def _():
        pltpu.sync_copy(zeros_hbm.at[pl.ds(tbase, ROWS_TAIL)],
                        acc_sh.at[pl.ds(tbase, ROWS_TAIL)])


def _acc_epilogue(out0_hbm, out1_hbm, acc_sh, c, s):
    plsc.subcore_barrier()
    rbase = s * ROWS_PER_SUB
    tbase = NS * ROWS_PER_SUB

    def _store(out_hbm):
        pltpu.sync_copy(acc_sh.at[pl.ds(rbase, ROWS_PER_SUB)],
                        out_hbm.at[pl.ds(rbase, ROWS_PER_SUB)])

        @pl.when(s == 0)
        def _():
            pltpu.sync_copy(acc_sh.at[pl.ds(tbase, ROWS_TAIL)],
                            out_hbm.at[pl.ds(tbase, ROWS_TAIL)])

    @pl.when(c == 0)
    def _():
        _store(out0_hbm)

    @pl.when(c == 1)
    def _():
        _store(out1_hbm)


def _make_segsum(width):
    @functools.partial(
        pl.kernel,
        out_type=[jax.ShapeDtypeStruct((N_NODES, width), jnp.float32),
                  jax.ShapeDtypeStruct((N_NODES, width), jnp.float32)],
        mesh=_mesh,
        scratch_types=(
            [pltpu.VMEM((NCHUNK, CHUNK), jnp.int32),
             pltpu.VMEM((NCHUNK, CHUNK), jnp.int32)]
            + [pltpu.VMEM((CHUNK, width), jnp.float32)] * NBUF
            + [pltpu.VMEM_SHARED((N_NODES, width), jnp.float32)]
            + [pltpu.SemaphoreType.DMA] * (1 + 2 * NBUF)
        ),
        compiler_params=_sc_params,
    )
    def seg(vals_hbm, eview_hbm, zeros_hbm, out0_hbm, out1_hbm,
            sidx, didx, *rest):
        bufs = rest[:NBUF]
        acc_sh = rest[NBUF]
        isem = rest[NBUF + 1]
        gsems = rest[NBUF + 2:NBUF + 2 + NBUF]
        ssems = rest[NBUF + 2 + NBUF:]
        c = lax.axis_index("c")
        s = lax.axis_index("s")
        w = c * NS + s
        pltpu.async_copy(eview_hbm.at[0, w], sidx, isem)
        pltpu.async_copy(eview_hbm.at[1, w], didx, isem)
        _zero_acc(zeros_hbm, acc_sh, s)
        pltpu.make_async_copy(eview_hbm.at[0, w], sidx, isem).wait()
        pltpu.make_async_copy(eview_hbm.at[1, w], didx, isem).wait()
        plsc.subcore_barrier()

        def g_desc(i, b):
            return pltpu.make_async_copy(vals_hbm.at[sidx.at[i]], bufs[b],
                                         gsems[b])

        def s_desc(i, b):
            return pltpu.make_async_copy(bufs[b], acc_sh.at[didx.at[i]],
                                         ssems[b])

        for b in range(NBUF):
            g_desc(b, b).start()

        @pl.loop(0, NCHUNK, step=NBUF)
        def _(i):
            for b in range(NBUF):
                g_desc(i + b, b).wait()
                pltpu.async_copy(bufs[b], acc_sh.at[didx.at[i + b]],
                                 ssems[b], add=True)
            for b in range(NBUF):
                s_desc(i + b, b).wait()
                nj = i + b + NBUF

                @pl.when(nj < NCHUNK)
                def _():
                    g_desc(nj, b).start()

        _acc_epilogue(out0_hbm, out1_hbm, acc_sh, c, s)

    return seg


_sc_seg16 = _make_segsum(HIDDEN)


@functools.partial(
    pl.kernel,
    out_type=[jax.ShapeDtypeStruct((N_NODES, HIDDEN), jnp.float32),
              jax.ShapeDtypeStruct((N_NODES, HIDDEN), jnp.float32)],
    mesh=_mesh,
    scratch_types=(
        [pltpu.VMEM((DCHUNK,), jnp.int32)] * DEG_FIRE
        + [pltpu.VMEM((DCHUNK, HIDDEN), jnp.float32),
           pltpu.VMEM_SHARED((N_NODES, HIDDEN), jnp.float32)]
        + [pltpu.SemaphoreType.DMA] * (2 * DEG_FIRE)
    ),
    compiler_params=_sc_params,
)
def _sc_deg(ones_hbm, edge_hbm, zeros_hbm, out0_hbm, out1_hbm, *rest):
    idxb = rest[:DEG_FIRE]
    ones_v = rest[DEG_FIRE]
    acc_sh = rest[DEG_FIRE + 1]
    isems = rest[DEG_FIRE + 2:DEG_FIRE + 2 + DEG_FIRE]
    ssems = rest[DEG_FIRE + 2 + DEG_FIRE:]
    c = lax.axis_index("c")
    s = lax.axis_index("s")
    w = c * NS + s
    ebase = w * EPW

    def i_desc(i, b):
        return pltpu.make_async_copy(
            edge_hbm.at[1, pl.ds(ebase + i * DCHUNK, DCHUNK)], idxb[b],
            isems[b])

    def s_desc(b):
        return pltpu.make_async_copy(ones_v, acc_sh.at[idxb[b]], ssems[b])

    for b in range(DEG_FIRE):
        i_desc(b, b).start()
    _zero_acc(zeros_hbm, acc_sh, s)
    pltpu.sync_copy(ones_hbm, ones_v)
    plsc.subcore_barrier()

    @pl.loop(0, DNCHUNK, step=DEG_FIRE)
    def _(i):
        for b in range(DEG_FIRE):
            i_desc(i + b, b).wait()
            pltpu.async_copy(ones_v, acc_sh.at[idxb[b]], ssems[b],
                             add=True)
        for b in range(DEG_FIRE):
            s_desc(b).wait()
            nj = i + b + DEG_FIRE

            @pl.when(nj < DNCHUNK)
            def _():
                i_desc(nj, b).start()

    _acc_epilogue(out0_hbm, out1_hbm, acc_sh, c, s)


def _tc_matmul1(xp, W1bd):
    def body(x_ref, w_ref, o_ref):
        o_ref[...] = jnp.dot(x_ref[...], w_ref[...], precision=_HIGH)
    return pl.pallas_call(
        body,
        out_shape=jax.ShapeDtypeStruct((NP, PACK * HIDDEN), jnp.float32),
    )(xp, W1bd)


def _tc_scale1(deg0, deg1, h1p):
    def body(d0_ref, d1_ref, h1_ref, dis_ref, h1s_ref):
        deg = (d0_ref[...].reshape(NP, PACK * HIDDEN)
               + d1_ref[...].reshape(NP, PACK * HIDDEN) + 1.0)
        dis = jax.lax.rsqrt(deg)
        dis_ref[...] = dis
        h1s_ref[...] = (dis * h1_ref[...]).reshape(FLAT)
    return pl.pallas_call(
        body,
        out_shape=(
            jax.ShapeDtypeStruct((NP, PACK * HIDDEN), jnp.float32),
            jax.ShapeDtypeStruct((FLAT,), jnp.float32),
        ),
    )(deg0, deg1, h1p)


def _tc_mid(s10, s11, h1p, disp, b1t, W2bd):
    def body(s0_ref, s1_ref, h1_ref, dis_ref, b1_ref, w2_ref, h2s_ref):
        dis = dis_ref[...]
        sv = (s0_ref[...].reshape(NP, PACK * HIDDEN)
              + s1_ref[...].reshape(NP, PACK * HIDDEN))
        out1 = dis * sv + dis * dis * h1_ref[...] + b1_ref[...][None, :]
        r = jnp.maximum(out1, 0.0)
        h2p = jnp.dot(r, w2_ref[...], precision=_HIGH)
        h2s_ref[...] = (dis * h2p).reshape(FLAT)
    return pl.pallas_call(
        body,
        out_shape=jax.ShapeDtypeStruct((FLAT,), jnp.float32),
    )(s10, s11, h1p, disp, b1t, W2bd)


def _tc_final(s20, s21, h2s, disp, b2t):
    def body(s0_ref, s1_ref, h2s_ref, dis_ref, b2_ref, o_ref):
        dis = dis_ref[...]
        sv = (s0_ref[...].reshape(NP, PACK * HIDDEN)
              + s1_ref[...].reshape(NP, PACK * HIDDEN))
        h2sp = h2s_ref[...].reshape(NP, PACK * HIDDEN)
        o_ref[...] = (dis * sv + dis * h2sp
                      + b2_ref[...][None, :]).reshape(FLAT)
    return pl.pallas_call(
        body,
        out_shape=jax.ShapeDtypeStruct((FLAT,), jnp.float32),
    )(s20, s21, h2s, disp, b2t)


def _blockdiag8(W):
    k, m = W.shape
    return (jnp.eye(PACK, dtype=W.dtype)[:, None, :, None]
            * W[None, :, None, :]).reshape(PACK * k, PACK * m)


def kernel(x, edge_index, W1, b1, W2, b2):
    eview = edge_index.reshape(2, NW, NCHUNK, CHUNK)
    zeros16 = jnp.zeros((N_NODES, HIDDEN), jnp.float32)
    ones_blk = jnp.ones((DCHUNK, HIDDEN), jnp.float32)
    xp = x.reshape(NP, PACK * D_FEAT)
    W1bd = _blockdiag8(W1)
    W2p = jnp.pad(W2, ((0, 0), (0, HIDDEN - N_ACTIONS)))
    W2bd = _blockdiag8(W2p)
    b1t = jnp.tile(b1, PACK)
    b2t = jnp.tile(jnp.pad(b2, (0, HIDDEN - N_ACTIONS)), PACK)

    deg0, deg1 = _sc_deg(ones_blk, edge_index, zeros16)
    h1p = _tc_matmul1(xp, W1bd)
    disp, h1s = _tc_scale1(deg0.reshape(FLAT), deg1.reshape(FLAT), h1p)
    s10, s11 = _sc_seg16(h1s.reshape(N_NODES, HIDDEN), eview, zeros16)
    h2s = _tc_mid(s10.reshape(FLAT), s11.reshape(FLAT), h1p, disp, b1t, W2bd)
    s20, s21 = _sc_seg16(h2s.reshape(N_NODES, HIDDEN), eview, zeros16)
    out = _tc_final(s20.reshape(FLAT), s21.reshape(FLAT), h2s, disp, b2t)
    return out.reshape(N_NODES, HIDDEN)[:, :N_ACTIONS]


# speedup vs baseline: 1.0662x; 1.0662x over previous
import functools

import jax
import jax.numpy as jnp
from jax import lax
from jax.experimental import pallas as pl
from jax.experimental.pallas import tpu as pltpu
from jax.experimental.pallas import tpu_sc as plsc

N_NODES = 10000
D_FEAT = 128
HIDDEN = 16
N_ACTIONS = 4
N_EDGES = 320000

NC = 2
NS = 16
NW = NC * NS
EPW = N_EDGES // NW
CHUNK = 125
NCHUNK = EPW // CHUNK
NBUF = 10
DCHUNK = 80
DNCHUNK = EPW // DCHUNK
DEG_FIRE = 10
ROWS_PER_SUB = 624
ROWS_TAIL = N_NODES - NS * ROWS_PER_SUB

PACK = 8
NP = N_NODES // PACK
FLAT = N_NODES * HIDDEN

_HIGH = jax.lax.Precision.HIGHEST

_mesh = plsc.VectorSubcoreMesh(core_axis_name="c", subcore_axis_name="s")
_sc_params = pltpu.CompilerParams(use_tc_tiling_on_sc=False)


def _zero_acc(zeros_hbm, acc_sh, s):
    rbase = s * ROWS_PER_SUB
    tbase = NS * ROWS_PER_SUB
    pltpu.sync_copy(zeros_hbm.at[pl.ds(rbase, ROWS_PER_SUB)],
                    acc_sh.at[pl.ds(rbase, ROWS_PER_SUB)])

    @pl.when(s == 0)
    def _():
        pltpu.sync_copy(zeros_hbm.at[pl.ds(tbase, ROWS_TAIL)],
                        acc_sh.at[pl.ds(tbase, ROWS_TAIL)])


def _acc_epilogue(out0_hbm, out1_hbm, acc_sh, c, s):
    plsc.subcore_barrier()
    rbase = s * ROWS_PER_SUB
    tbase = NS * ROWS_PER_SUB

    def _store(out_hbm):
        pltpu.sync_copy(acc_sh.at[pl.ds(rbase, ROWS_PER_SUB)],
                        out_hbm.at[pl.ds(rbase, ROWS_PER_SUB)])

        @pl.when(s == 0)
        def _():
            pltpu.sync_copy(acc_sh.at[pl.ds(tbase, ROWS_TAIL)],
                            out_hbm.at[pl.ds(tbase, ROWS_TAIL)])

    @pl.when(c == 0)
    def _():
        _store(out0_hbm)

    @pl.when(c == 1)
    def _():
        _store(out1_hbm)


def _make_segsum(width):
    @functools.partial(
        pl.kernel,
        out_type=[jax.ShapeDtypeStruct((N_NODES, width), jnp.float32),
                  jax.ShapeDtypeStruct((N_NODES, width), jnp.float32)],
        mesh=_mesh,
        scratch_types=(
            [pltpu.VMEM((NCHUNK, CHUNK), jnp.int32),
             pltpu.VMEM((NCHUNK, CHUNK), jnp.int32)]
            + [pltpu.VMEM((CHUNK, width), jnp.float32)] * NBUF
            + [pltpu.VMEM_SHARED((N_NODES, width), jnp.float32)]
            + [pltpu.SemaphoreType.DMA] * (1 + 2 * NBUF)
        ),
        compiler_params=_sc_params,
    )
    def seg(vals_hbm, eview_hbm, zeros_hbm, out0_hbm, out1_hbm,
            sidx, didx, *rest):
        bufs = rest[:NBUF]
        acc_sh = rest[NBUF]
        isem = rest[NBUF + 1]
        gsems = rest[NBUF + 2:NBUF + 2 + NBUF]
        ssems = rest[NBUF + 2 + NBUF:]
        c = lax.axis_index("c")
        s = lax.axis_index("s")
        w = c * NS + s
        pltpu.async_copy(eview_hbm.at[0, w], sidx, isem)
        pltpu.async_copy(eview_hbm.at[1, w], didx, isem)
        _zero_acc(zeros_hbm, acc_sh, s)
        pltpu.make_async_copy(eview_hbm.at[0, w], sidx, isem).wait()
        pltpu.make_async_copy(eview_hbm.at[1, w], didx, isem).wait()
        plsc.subcore_barrier()

        def g_desc(i, b):
            return pltpu.make_async_copy(vals_hbm.at[sidx.at[i]], bufs[b],
                                         gsems[b])

        def s_desc(i, b):
            return pltpu.make_async_copy(bufs[b], acc_sh.at[didx.at[i]],
                                         ssems[b])

        for b in range(NBUF):
            g_desc(b, b).start()

        @pl.loop(0, NCHUNK, step=NBUF)
        def _(i):
            for b in range(NBUF):
                g_desc(i + b, b).wait()
                pltpu.async_copy(bufs[b], acc_sh.at[didx.at[i + b]],
                                 ssems[b], add=True)
            for b in range(NBUF):
                s_desc(i + b, b).wait()
                nj = i + b + NBUF

                @pl.when(nj < NCHUNK)
                def _():
                    g_desc(nj, b).start()

        _acc_epilogue(out0_hbm, out1_hbm, acc_sh, c, s)

    return seg


_sc_seg16 = _make_segsum(HIDDEN)


@functools.partial(
    pl.kernel,
    out_type=[jax.ShapeDtypeStruct((N_NODES, HIDDEN), jnp.float32),
              jax.ShapeDtypeStruct((N_NODES, HIDDEN), jnp.float32)],
    mesh=_mesh,
    scratch_types=[
        pltpu.VMEM((NCHUNK, CHUNK), jnp.int32),
        pltpu.VMEM((CHUNK, HIDDEN), jnp.float32),
        pltpu.VMEM_SHARED((N_NODES, HIDDEN), jnp.float32),
        pltpu.SemaphoreType.DMA,
        pltpu.SemaphoreType.DMA,
    ],
    compiler_params=_sc_params,
)
def _sc_deg(ones_hbm, eview_hbm, zeros_hbm, out0_hbm, out1_hbm,
            didx, ones_v, acc_sh, isem, dsem):
    c = lax.axis_index("c")
    s = lax.axis_index("s")
    w = c * NS + s
    pltpu.async_copy(eview_hbm.at[1, w], didx, isem)
    _zero_acc(zeros_hbm, acc_sh, s)
    pltpu.sync_copy(ones_hbm, ones_v)
    pltpu.make_async_copy(eview_hbm.at[1, w], didx, isem).wait()
    plsc.subcore_barrier()

    @pl.loop(0, NCHUNK, step=DEG_FIRE)
    def _(i):
        for b in range(DEG_FIRE):
            pltpu.async_copy(ones_v, acc_sh.at[didx.at[i + b]], dsem,
                             add=True)
        for b in range(DEG_FIRE):
            pltpu.make_async_copy(ones_v, acc_sh.at[didx.at[i + b]],
                                  dsem).wait()

    _acc_epilogue(out0_hbm, out1_hbm, acc_sh, c, s)


def _tc_matmul1(xp, W1bd):
    def body(x_ref, w_ref, o_ref):
        o_ref[...] = jnp.dot(x_ref[...], w_ref[...], precision=_HIGH)
    return pl.pallas_call(
        body,
        out_shape=jax.ShapeDtypeStruct((NP, PACK * HIDDEN), jnp.float32),
    )(xp, W1bd)


def _tc_scale1(deg0, deg1, h1p):
    def body(d0_ref, d1_ref, h1_ref, dis_ref, h1s_ref):
        deg = (d0_ref[...].reshape(NP, PACK * HIDDEN)
               + d1_ref[...].reshape(NP, PACK * HIDDEN) + 1.0)
        dis = jax.lax.rsqrt(deg)
        dis_ref[...] = dis
        h1s_ref[...] = (dis * h1_ref[...]).reshape(FLAT)
    return pl.pallas_call(
        body,
        out_shape=(
            jax.ShapeDtypeStruct((NP, PACK * HIDDEN), jnp.float32),
            jax.ShapeDtypeStruct((FLAT,), jnp.float32),
        ),
    )(deg0, deg1, h1p)


def _tc_mid(s10, s11, h1p, disp, b1t, W2bd):
    def body(s0_ref, s1_ref, h1_ref, dis_ref, b1_ref, w2_ref, h2s_ref):
        dis = dis_ref[...]
        sv = (s0_ref[...].reshape(NP, PACK * HIDDEN)
              + s1_ref[...].reshape(NP, PACK * HIDDEN))
        out1 = dis * sv + dis * dis * h1_ref[...] + b1_ref[...][None, :]
        r = jnp.maximum(out1, 0.0)
        h2p = jnp.dot(r, w2_ref[...], precision=_HIGH)
        h2s_ref[...] = (dis * h2p).reshape(FLAT)
    return pl.pallas_call(
        body,
        out_shape=jax.ShapeDtypeStruct((FLAT,), jnp.float32),
    )(s10, s11, h1p, disp, b1t, W2bd)


def _tc_final(s20, s21, h2s, disp, b2t):
    def body(s0_ref, s1_ref, h2s_ref, dis_ref, b2_ref, o_ref):
        dis = dis_ref[...]
        sv = (s0_ref[...].reshape(NP, PACK * HIDDEN)
              + s1_ref[...].reshape(NP, PACK * HIDDEN))
        h2sp = h2s_ref[...].reshape(NP, PACK * HIDDEN)
        o_ref[...] = (dis * sv + dis * h2sp
                      + b2_ref[...][None, :]).reshape(FLAT)
    return pl.pallas_call(
        body,
        out_shape=jax.ShapeDtypeStruct((FLAT,), jnp.float32),
    )(s20, s21, h2s, disp, b2t)


def _blockdiag8(W):
    k, m = W.shape
    return (jnp.eye(PACK, dtype=W.dtype)[:, None, :, None]
            * W[None, :, None, :]).reshape(PACK * k, PACK * m)


def kernel(x, edge_index, W1, b1, W2, b2):
    eview = edge_index.reshape(2, NW, NCHUNK, CHUNK)
    zeros16 = jnp.zeros((N_NODES, HIDDEN), jnp.float32)
    ones_blk = jnp.ones((CHUNK, HIDDEN), jnp.float32)
    xp = x.reshape(NP, PACK * D_FEAT)
    W1bd = _blockdiag8(W1)
    W2p = jnp.pad(W2, ((0, 0), (0, HIDDEN - N_ACTIONS)))
    W2bd = _blockdiag8(W2p)
    b1t = jnp.tile(b1, PACK)
    b2t = jnp.tile(jnp.pad(b2, (0, HIDDEN - N_ACTIONS)), PACK)

    deg0, deg1 = _sc_deg(ones_blk, eview, zeros16)
    h1p = _tc_matmul1(xp, W1bd)
    disp, h1s = _tc_scale1(deg0.reshape(FLAT), deg1.reshape(FLAT), h1p)
    s10, s11 = _sc_seg16(h1s.reshape(N_NODES, HIDDEN), eview, zeros16)
    h2s = _tc_mid(s10.reshape(FLAT), s11.reshape(FLAT), h1p, disp, b1t, W2bd)
    s20, s21 = _sc_seg16(h2s.reshape(N_NODES, HIDDEN), eview, zeros16)
    out = _tc_final(s20.reshape(FLAT), s21.reshape(FLAT), h2s, disp, b2t)
    return out.reshape(N_NODES, HIDDEN)[:, :N_ACTIONS]
